# baseline (speedup 1.0000x reference)
import functools

import jax
import jax.numpy as jnp
from jax import lax
from jax.experimental import pallas as pl
from jax.experimental.pallas import tpu as pltpu
from jax.experimental.pallas import tpu_sc as plsc

_N = 1000
_B = 16384
_NC = 2
_NS = 16
_LANES = 16
_NW = _NC * _NS
_BPW = _B // _NW
_CHUNKS = _BPW // _LANES
_ROWS = _BPW // 128


def _lse_body(wa_ref, wba_ref, g_ref):
    wba = wba_ref[...]
    m_b = jnp.max(wba, axis=1, keepdims=True)
    s_b = jnp.sum(jnp.exp(wba - m_b), axis=1, keepdims=True)
    lse_b = jnp.log(s_b) + m_b
    wa = wa_ref[...]
    m_a = jnp.max(wa)
    s_a = jnp.sum(jnp.exp(wa - m_a))
    lse_a = jnp.log(s_a) + m_a
    g_ref[...] = wa - (lse_b + lse_a)


def _reg_take(x, idx):
    dnums = lax.GatherDimensionNumbers(
        offset_dims=(), collapsed_slice_dims=(0,), start_index_map=(0,))
    return lax.gather(x, idx[:, None], dnums, (1,),
                      mode=lax.GatherScatterMode.PROMISE_IN_BOUNDS)


def _sc_body(g_hbm, wba_hbm, iab_hbm, out_hbm,
             iab_v, f_v, gi_v, wv, gv, out_v, sem):
    wid = lax.axis_index("s") * _NC + lax.axis_index("c")
    base = wid * _BPW
    pltpu.sync_copy(iab_hbm.at[pl.ds(2 * base, 2 * _BPW)], iab_v)
    lane = lax.iota(jnp.int32, _LANES)
    even = (lane & 7) * 2
    lo = lane < 8
    for j in range(_CHUNKS):
        r, col = j // 8, (j % 8) * _LANES
        v1 = iab_v[pl.ds(j * 2 * _LANES, _LANES)]
        v2 = iab_v[pl.ds(j * 2 * _LANES + _LANES, _LANES)]
        a16 = jnp.where(lo, _reg_take(v1, even), _reg_take(v2, even))
        c16 = jnp.where(lo, _reg_take(v1, even + 1), _reg_take(v2, even + 1))
        f_v[r, pl.ds(col, _LANES)] = a16 * _N + c16
        gi_v[r, pl.ds(col, _LANES)] = a16
    copies = []
    for r in range(_ROWS):
        copies.append(pltpu.async_copy(wba_hbm.at[f_v.at[r]], wv.at[r], sem))
        copies.append(pltpu.async_copy(g_hbm.at[gi_v.at[r]], gv.at[r], sem))
    for cp in copies:
        cp.wait()
    for j in range(_CHUNKS):
        r, col = j // 8, (j % 8) * _LANES
        out_v[pl.ds(j * _LANES, _LANES)] = (
            wv[r, pl.ds(col, _LANES)] + gv[r, pl.ds(col, _LANES)]
        )
    pltpu.sync_copy(out_v, out_hbm.at[pl.ds(base, _BPW)])


@functools.cache
def _sc_gather():
  return pl.kernel(
    _sc_body,
    out_type=jax.ShapeDtypeStruct((_B,), jnp.float32),
    mesh=plsc.VectorSubcoreMesh(core_axis_name="c", subcore_axis_name="s",
                                num_cores=_NC, num_subcores=_NS),
    scratch_types=[
        pltpu.VMEM((2 * _BPW,), jnp.int32),
        pltpu.VMEM((_ROWS, 128), jnp.int32),
        pltpu.VMEM((_ROWS, 128), jnp.int32),
        pltpu.VMEM((_ROWS, 128), jnp.float32),
        pltpu.VMEM((_ROWS, 128), jnp.float32),
        pltpu.VMEM((_BPW,), jnp.float32),
        pltpu.SemaphoreType.DMA,
    ],
  )


def kernel(inputs, w_A, w_BA):
    g_col = pl.pallas_call(
        _lse_body,
        out_shape=jax.ShapeDtypeStruct((_N, 1), jnp.float32),
    )(w_A.reshape(_N, 1), w_BA)
    return _sc_gather()(g_col.reshape(-1), w_BA.reshape(-1),
                        inputs.astype(jnp.int32).reshape(-1))


# speedup vs baseline: 1.2436x; 1.2436x over previous
import functools

import jax
import jax.numpy as jnp
from jax import lax
from jax.experimental import pallas as pl
from jax.experimental.pallas import tpu as pltpu
from jax.experimental.pallas import tpu_sc as plsc

_N = 1000
_B = 16384
_NC = 2
_NS = 16
_LANES = 16
_NW = _NC * _NS
_BPW = _B // _NW
_CHUNKS = _BPW // _LANES
_ROWS = _BPW // 128


def _lse_body(wa_ref, wba_ref, g_ref):
    wba = wba_ref[...]
    m_b = jnp.max(wba, axis=1, keepdims=True)
    s_b = jnp.sum(jnp.exp(wba - m_b), axis=1, keepdims=True)
    lse_b = jnp.log(s_b) + m_b
    wa = wa_ref[...]
    m_a = jnp.max(wa)
    s_a = jnp.sum(jnp.exp(wa - m_a))
    lse_a = jnp.log(s_a) + m_a
    g_ref[...] = jnp.broadcast_to(wa - (lse_b + lse_a), (_N, 128))


def _reg_take(x, idx):
    dnums = lax.GatherDimensionNumbers(
        offset_dims=(), collapsed_slice_dims=(0,), start_index_map=(0,))
    return lax.gather(x, idx[:, None], dnums, (1,),
                      mode=lax.GatherScatterMode.PROMISE_IN_BOUNDS)


def _sc_body(g_hbm, wba_hbm, iab_hbm, out_hbm,
             iab_v, f_v, gi_v, wv, gv, out_v, sem):
    wid = lax.axis_index("s") * _NC + lax.axis_index("c")
    base = wid * _BPW
    pltpu.sync_copy(iab_hbm.at[pl.ds(2 * base, 2 * _BPW)], iab_v)
    lane = lax.iota(jnp.int32, _LANES)
    even = (lane & 7) * 2
    lo = lane < 8
    for j in range(_CHUNKS):
        r, col = j // 8, (j % 8) * _LANES
        v1 = iab_v[pl.ds(j * 2 * _LANES, _LANES)]
        v2 = iab_v[pl.ds(j * 2 * _LANES + _LANES, _LANES)]
        a16 = jnp.where(lo, _reg_take(v1, even), _reg_take(v2, even))
        c16 = jnp.where(lo, _reg_take(v1, even + 1), _reg_take(v2, even + 1))
        f_v[r, pl.ds(col, _LANES)] = a16 * _N + c16
        gi_v[r, pl.ds(col, _LANES)] = a16 * 128
    copies = []
    for r in range(_ROWS):
        copies.append(pltpu.async_copy(wba_hbm.at[f_v.at[r]], wv.at[r], sem))
        copies.append(pltpu.async_copy(g_hbm.at[gi_v.at[r]], gv.at[r], sem))
    for cp in copies:
        cp.wait()
    for j in range(_CHUNKS):
        r, col = j // 8, (j % 8) * _LANES
        out_v[pl.ds(j * _LANES, _LANES)] = (
            wv[r, pl.ds(col, _LANES)] + gv[r, pl.ds(col, _LANES)]
        )
    pltpu.sync_copy(out_v, out_hbm.at[pl.ds(base, _BPW)])


@functools.cache
def _sc_gather():
  return pl.kernel(
    _sc_body,
    out_type=jax.ShapeDtypeStruct((_B,), jnp.float32),
    mesh=plsc.VectorSubcoreMesh(core_axis_name="c", subcore_axis_name="s",
                                num_cores=_NC, num_subcores=_NS),
    scratch_types=[
        pltpu.VMEM((2 * _BPW,), jnp.int32),
        pltpu.VMEM((_ROWS, 128), jnp.int32),
        pltpu.VMEM((_ROWS, 128), jnp.int32),
        pltpu.VMEM((_ROWS, 128), jnp.float32),
        pltpu.VMEM((_ROWS, 128), jnp.float32),
        pltpu.VMEM((_BPW,), jnp.float32),
        pltpu.SemaphoreType.DMA,
    ],
  )


def kernel(inputs, w_A, w_BA):
    g_col = pl.pallas_call(
        _lse_body,
        out_shape=jax.ShapeDtypeStruct((_N, 128), jnp.float32),
    )(w_A.reshape(_N, 1), w_BA)
    return _sc_gather()(g_col.reshape(-1), w_BA.reshape(-1),
                        inputs.astype(jnp.int32).reshape(-1))


# speedup vs baseline: 1.6749x; 1.3468x over previous
import functools

import jax
import jax.numpy as jnp
from jax import lax
from jax.experimental import pallas as pl
from jax.experimental.pallas import tpu as pltpu
from jax.experimental.pallas import tpu_sc as plsc

_N = 1000
_NP = 1024
_B = 16384
_NC = 2
_NS = 16
_LANES = 16
_NW = _NC * _NS
_BPW = _B // _NW
_CHUNKS = _BPW // _LANES
_ROWS = _BPW // 128


def _lse_body(wa_ref, wba_ref, wp_ref):
    wba = wba_ref[...]
    m_b = jnp.max(wba, axis=1, keepdims=True)
    s_b = jnp.sum(jnp.exp(wba - m_b), axis=1, keepdims=True)
    lse_b = jnp.log(s_b) + m_b
    wa = wa_ref[...]
    wa_col = wa[:, 0:1]
    m_a = jnp.max(wa_col)
    s_a = jnp.sum(jnp.exp(wa_col - m_a))
    lse_a = jnp.log(s_a) + m_a
    wp_ref[:, : _N] = wba + (wa_col - (lse_b + lse_a))


def _sc_body(wp_hbm, ia_hbm, ib_hbm, out_hbm, ia_v, ib_v, f_v, wv, sem):
    wid = lax.axis_index("s") * _NC + lax.axis_index("c")
    base = wid * _BPW
    pltpu.sync_copy(ia_hbm.at[pl.ds(base, _BPW)], ia_v)
    pltpu.sync_copy(ib_hbm.at[pl.ds(base, _BPW)], ib_v)
    for j in range(_CHUNKS):
        r, col = j // 8, (j % 8) * _LANES
        a16 = ia_v[pl.ds(j * _LANES, _LANES)]
        c16 = ib_v[pl.ds(j * _LANES, _LANES)]
        f_v[r, pl.ds(col, _LANES)] = a16 * _NP + c16
    copies = []
    for r in range(_ROWS):
        copies.append(
            pltpu.async_copy(wp_hbm.at[f_v.at[r]], wv.at[pl.ds(r * 128, 128)], sem))
    for cp in copies:
        cp.wait()
    pltpu.sync_copy(wv, out_hbm.at[pl.ds(base, _BPW)])


@functools.cache
def _sc_gather():
  return pl.kernel(
    _sc_body,
    out_type=jax.ShapeDtypeStruct((_B,), jnp.float32),
    mesh=plsc.VectorSubcoreMesh(core_axis_name="c", subcore_axis_name="s",
                                num_cores=_NC, num_subcores=_NS),
    scratch_types=[
        pltpu.VMEM((_BPW,), jnp.int32),
        pltpu.VMEM((_BPW,), jnp.int32),
        pltpu.VMEM((_ROWS, 128), jnp.int32),
        pltpu.VMEM((_BPW,), jnp.float32),
        pltpu.SemaphoreType.DMA,
    ],
  )


def kernel(inputs, w_A, w_BA):
    idx = inputs.astype(jnp.int32)
    idx_a = idx[:, 0]
    idx_b = idx[:, 1]
    wa_b = jnp.broadcast_to(w_A[:, None], (_N, 128))
    wp = pl.pallas_call(
        _lse_body,
        out_shape=jax.ShapeDtypeStruct((_N, _NP), jnp.float32),
    )(wa_b, w_BA)
    return _sc_gather()(wp.reshape(-1), idx_a, idx_b)
